# initial kernel scaffold (unmeasured)
import jax
import jax.numpy as jnp
from jax import lax
from jax.experimental import pallas as pl
from jax.experimental.pallas import tpu as pltpu

N_DEV = 16
M = 512
D = 512
CM = M // N_DEV
EPS = 1e-6


def kernel(partial, resid, gamma):
    partial = partial.reshape(M, D)

    def body(partial_ref, resid_ref, gamma_ref, out_ref,
             rs_buf, send1, recv1, send2, recv2):
        my = lax.axis_index("i")

        rdmas1 = []
        for k in range(1, N_DEV):
            peer = lax.rem(my + k, N_DEV)
            rdma = pltpu.make_async_remote_copy(
                src_ref=partial_ref.at[pl.ds(peer * CM, CM), :],
                dst_ref=rs_buf.at[k],
                send_sem=send1.at[k],
                recv_sem=recv1.at[k],
                device_id=(peer,),
                device_id_type=pl.DeviceIdType.MESH,
            )
            rdma.start()
            rdmas1.append(rdma)

        rs_buf[0] = partial_ref[pl.ds(my * CM, CM), :]

        for rdma in rdmas1:
            rdma.wait_recv()

        y = jnp.sum(rs_buf[...], axis=0) + resid_ref[pl.ds(my * CM, CM), :]
        rms = jnp.sqrt(jnp.mean(y * y, axis=-1, keepdims=True) + EPS)
        out_ref[pl.ds(my * CM, CM), :] = y / rms * gamma_ref[...]

        for rdma in rdmas1:
            rdma.wait_send()

        rdmas2 = []
        for k in range(1, N_DEV):
            peer = lax.rem(my + k, N_DEV)
            rdma = pltpu.make_async_remote_copy(
                src_ref=out_ref.at[pl.ds(my * CM, CM), :],
                dst_ref=out_ref.at[pl.ds(my * CM, CM), :],
                send_sem=send2.at[k],
                recv_sem=recv2.at[k],
                device_id=(peer,),
                device_id_type=pl.DeviceIdType.MESH,
            )
            rdma.start()
            rdmas2.append(rdma)

        for rdma in rdmas2:
            rdma.wait_recv()
            rdma.wait_send()

    return pl.pallas_call(
        body,
        out_shape=jax.ShapeDtypeStruct((M, D), jnp.float32),
        in_specs=[
            pl.BlockSpec(memory_space=pltpu.VMEM),
            pl.BlockSpec(memory_space=pltpu.VMEM),
            pl.BlockSpec(memory_space=pltpu.VMEM),
        ],
        out_specs=pl.BlockSpec(memory_space=pltpu.VMEM),
        scratch_shapes=[
            pltpu.VMEM((N_DEV, CM, D), jnp.float32),
            pltpu.SemaphoreType.DMA((N_DEV,)),
            pltpu.SemaphoreType.DMA((N_DEV,)),
            pltpu.SemaphoreType.DMA((N_DEV,)),
            pltpu.SemaphoreType.DMA((N_DEV,)),
        ],
        compiler_params=pltpu.CompilerParams(collective_id=0),
    )(partial, resid, gamma)


# baseline (device time: 36516 ns/iter reference)
import jax
import jax.numpy as jnp
from jax import lax
from jax.experimental import pallas as pl
from jax.experimental.pallas import tpu as pltpu

N_DEV = 16
M = 512
D = 512
CM = M // N_DEV
EPS = 1e-6


def kernel(partial, resid, gamma):
    partial = partial.reshape(M, D)

    def body(partial_ref, resid_ref, gamma_ref, out_ref,
             rs_buf, send1, recv1, send2, recv2):
        my = lax.axis_index("i")

        rdmas1 = []
        for k in range(1, N_DEV):
            peer = lax.rem(my + k, N_DEV)
            rdma = pltpu.make_async_remote_copy(
                src_ref=partial_ref.at[pl.ds(peer * CM, CM), :],
                dst_ref=rs_buf.at[k],
                send_sem=send1.at[k],
                recv_sem=recv1.at[k],
                device_id=(peer,),
                device_id_type=pl.DeviceIdType.MESH,
            )
            rdma.start()
            rdmas1.append(rdma)

        rs_buf[0] = partial_ref[pl.ds(my * CM, CM), :]

        for rdma in rdmas1:
            rdma.wait_recv()

        y = jnp.sum(rs_buf[...], axis=0) + resid_ref[pl.ds(my * CM, CM), :]
        rms = jnp.sqrt(jnp.mean(y * y, axis=-1, keepdims=True) + EPS)
        out_ref[pl.ds(my * CM, CM), :] = y / rms * gamma_ref[...]

        for rdma in rdmas1:
            rdma.wait_send()

        rdmas2 = []
        for k in range(1, N_DEV):
            peer = lax.rem(my + k, N_DEV)
            rdma = pltpu.make_async_remote_copy(
                src_ref=out_ref.at[pl.ds(my * CM, CM), :],
                dst_ref=out_ref.at[pl.ds(my * CM, CM), :],
                send_sem=send2.at[k],
                recv_sem=recv2.at[k],
                device_id=(peer,),
                device_id_type=pl.DeviceIdType.MESH,
            )
            rdma.start()
            rdmas2.append(rdma)

        for rdma in rdmas2:
            rdma.wait_recv()
            rdma.wait_send()

    return pl.pallas_call(
        body,
        out_shape=jax.ShapeDtypeStruct((M, D), jnp.float32),
        in_specs=[
            pl.BlockSpec(memory_space=pltpu.VMEM),
            pl.BlockSpec(memory_space=pltpu.VMEM),
            pl.BlockSpec(memory_space=pltpu.VMEM),
        ],
        out_specs=pl.BlockSpec(memory_space=pltpu.VMEM),
        scratch_shapes=[
            pltpu.VMEM((N_DEV, CM, D), jnp.float32),
            pltpu.SemaphoreType.DMA((N_DEV,)),
            pltpu.SemaphoreType.DMA((N_DEV,)),
            pltpu.SemaphoreType.DMA((N_DEV,)),
            pltpu.SemaphoreType.DMA((N_DEV,)),
        ],
    )(partial, resid, gamma)
